# baseline (speedup 1.0000x reference)
import functools

import jax
import jax.numpy as jnp
import numpy as np
from jax.experimental import pallas as pl
from jax.experimental.pallas import tpu as pltpu

_EPS_BN = 1e-5
_K = 1024
_D = 128
_TILE = 512


def _bn(x):
    return x / np.sqrt(1.0 + _EPS_BN)


def _conv2d(x, w, b=None, stride=1, padding=0):
    out = jax.lax.conv_general_dilated(
        x, w, (stride, stride), [(padding, padding), (padding, padding)],
        dimension_numbers=('NCHW', 'OIHW', 'NCHW'))
    if b is not None:
        out = out + b[None, :, None, None]
    return out


def _conv_transpose2d(x, w, b, stride=2, padding=1):
    kh, kw = w.shape[2], w.shape[3]
    w_t = jnp.transpose(w[:, :, ::-1, ::-1], (1, 0, 2, 3))
    ph, pw = kh - 1 - padding, kw - 1 - padding
    out = jax.lax.conv_general_dilated(
        x, w_t, (1, 1), [(ph, ph), (pw, pw)], lhs_dilation=(stride, stride),
        dimension_numbers=('NCHW', 'OIHW', 'NCHW'))
    return out + b[None, :, None, None]


def _residual_stack(x, layers):
    for p in layers:
        h = jax.nn.relu(x)
        h = _conv2d(h, p['w1'], None, 1, 1)
        h = _bn(h)
        h = jax.nn.relu(h)
        h = _conv2d(h, p['w2'], None, 1, 0)
        h = _bn(h)
        x = x + h
    return jax.nn.relu(x)


def _conv2d_nhwc(x, w, b=None, stride=1, padding=0):
    out = jax.lax.conv_general_dilated(
        x, jnp.transpose(w, (2, 3, 1, 0)), (stride, stride),
        [(padding, padding), (padding, padding)],
        dimension_numbers=('NHWC', 'HWIO', 'NHWC'))
    if b is not None:
        out = out + b[None, None, None, :]
    return out


def _residual_stack_nhwc(x, layers):
    for p in layers:
        h = jax.nn.relu(x)
        h = _conv2d_nhwc(h, p['w1'], None, 1, 1)
        h = _bn(h)
        h = jax.nn.relu(h)
        h = _conv2d_nhwc(h, p['w2'], None, 1, 0)
        h = _bn(h)
        x = x + h
    return jax.nn.relu(x)


def _vq_body(n_total, n_steps, z_ref, mcbt_ref, c2_ref, q_ref,
             loss_ref, perp_ref, hist_ref, acc_ref):
    i = pl.program_id(0)

    @pl.when(i == 0)
    def _():
        hist_ref[...] = jnp.zeros_like(hist_ref)
        acc_ref[0, 0] = 0.0

    zt = z_ref[...]
    dist = jnp.dot(zt, mcbt_ref[...],
                   preferred_element_type=jnp.float32) + c2_ref[...]

    iota = jax.lax.broadcasted_iota(jnp.int32, dist.shape, 1)
    big = jnp.float32(jnp.inf)

    d0 = jnp.min(dist, axis=1, keepdims=True)
    i0 = jnp.min(jnp.where(dist == d0, iota, _K), axis=1, keepdims=True)
    dist1 = jnp.where(iota == i0, big, dist)
    d1 = jnp.min(dist1, axis=1, keepdims=True)
    i1 = jnp.min(jnp.where(dist1 == d1, iota, _K), axis=1, keepdims=True)
    dist2 = jnp.where(iota == i1, big, dist1)
    d2 = jnp.min(dist2, axis=1, keepdims=True)
    i2 = jnp.min(jnp.where(dist2 == d2, iota, _K), axis=1, keepdims=True)

    q_ref[...] = i0

    z2 = jnp.sum(zt * zt, axis=1, keepdims=True)
    acc_ref[0, 0] += jnp.sum(d0 + z2)

    onehot2 = (iota == i2).astype(jnp.float32)
    hist_ref[...] += jnp.sum(onehot2, axis=0, keepdims=True)

    @pl.when(i == n_steps - 1)
    def _():
        loss_ref[0, 0] = acc_ref[0, 0] * (0.25 / (n_total * _D))
        avg = hist_ref[...] / n_total
        perp_ref[0, 0] = jnp.exp(-jnp.sum(avg * jnp.log(avg + 1e-10)))


@functools.partial(jax.jit, static_argnames=('interpret',))
def _vq_pallas(z_flat, codebook, interpret=False):
    n_total = z_flat.shape[0]
    n_steps = n_total // _TILE
    mcbt = -2.0 * codebook.T
    c2 = jnp.sum(codebook * codebook, axis=1)[None, :]
    i0, loss, perp = pl.pallas_call(
        functools.partial(_vq_body, n_total, n_steps),
        grid=(n_steps,),
        in_specs=[
            pl.BlockSpec((_TILE, _D), lambda i: (i, 0)),
            pl.BlockSpec((_D, _K), lambda i: (0, 0)),
            pl.BlockSpec((1, _K), lambda i: (0, 0)),
        ],
        out_specs=[
            pl.BlockSpec((_TILE, 1), lambda i: (i, 0)),
            pl.BlockSpec(memory_space=pltpu.SMEM),
            pl.BlockSpec(memory_space=pltpu.SMEM),
        ],
        out_shape=[
            jax.ShapeDtypeStruct((n_total, 1), jnp.int32),
            jax.ShapeDtypeStruct((1, 1), jnp.float32),
            jax.ShapeDtypeStruct((1, 1), jnp.float32),
        ],
        scratch_shapes=[
            pltpu.VMEM((1, _K), jnp.float32),
            pltpu.SMEM((1, 1), jnp.float32),
        ],
        interpret=interpret,
    )(z_flat, mcbt, c2)
    return i0, loss[0, 0], perp[0, 0]


def _sc_gather(codebook, idx):
    from jax.experimental.pallas import tpu_sc as plsc
    info = plsc.get_sparse_core_info()
    nw = info.num_cores * info.num_subcores
    b_total = idx.shape[0]
    b_per_w = b_total // nw
    mesh = plsc.VectorSubcoreMesh(core_axis_name="c", subcore_axis_name="s")

    n_chunk = 4
    ch = b_per_w // n_chunk

    @functools.partial(
        pl.kernel, mesh=mesh,
        out_type=jax.ShapeDtypeStruct((b_total, _D), jnp.float32),
        scratch_types=[
            pltpu.VMEM((ch,), jnp.int32),
            pltpu.VMEM((ch, _D), jnp.float32),
            pltpu.SemaphoreType.DMA,
        ],
    )
    def k(table_hbm, idx_hbm, out_hbm, idx_v, rows_v, sem):
        wid = jax.lax.axis_index("s") * info.num_cores + jax.lax.axis_index("c")
        for j in range(n_chunk):
            base = wid * b_per_w + j * ch
            pltpu.sync_copy(idx_hbm.at[pl.ds(base, ch)], idx_v)
            pltpu.async_copy(table_hbm.at[idx_v], rows_v, sem).wait()
            pltpu.sync_copy(rows_v, out_hbm.at[pl.ds(base, ch)])

    return k(codebook, idx)


_DELTAS = tuple((dh, dw) for dh in (-1, 0, 1) for dw in (-1, 0, 1))
_KMAP = {(0, -1): 0, (0, 0): 2, (1, 0): 1, (1, 1): 3}
_T2TAB = (
    ((1, -1, 0), (0, 0, 2)),
    ((0, 0, 1), (1, 0, 3)),
    ((0, 0, 0), (1, 0, 2)),
    ((1, 0, 1), (0, 1, 3)),
)


def _upconv_masks():
    g = np.arange(50176)
    h = (g // 56) % 56
    w = g % 56
    cols = [h > 0, h < 55, w > 0, w < 55, np.ones(50176, bool),
            np.ones(50176, bool), np.ones(50176, bool), np.ones(50176, bool)]
    return np.stack(cols, axis=1).astype(np.float32)


def _t1_weights(t1_w):
    w_t = jnp.transpose(t1_w[:, :, ::-1, ::-1], (1, 0, 2, 3))
    mats = []
    for dh, dw in _DELTAS:
        wd = jnp.zeros((128, 256), jnp.float32)
        for rh in (0, 1):
            for rw in (0, 1):
                if (rh, dh) in _KMAP and (rw, dw) in _KMAP:
                    blk = (rh * 2 + rw) * 64
                    wd = wd.at[:, blk:blk + 64].set(
                        w_t[:, :, _KMAP[(rh, dh)], _KMAP[(rw, dw)]].T)
        mats.append(wd)
    return jnp.stack(mats)


def _t2_weights(t2_w):
    v_t = jnp.transpose(t2_w[:, :, ::-1, ::-1], (1, 0, 2, 3))
    mats = []
    for dh, dw in _DELTAS:
        vd = jnp.zeros((256, 48), jnp.float32)
        for ch in range(4):
            for cw in range(4):
                for (rho_h, dh2, vrh) in _T2TAB[ch]:
                    if dh2 != dh:
                        continue
                    for (rho_w, dw2, vrw) in _T2TAB[cw]:
                        if dw2 != dw:
                            continue
                        rblk = (rho_h * 2 + rho_w) * 64
                        cblk = (ch * 4 + cw) * 3
                        vd = vd.at[rblk:rblk + 64, cblk:cblk + 3].add(
                            v_t[:, :, vrh, vrw].T)
        mats.append(vd)
    return jnp.stack(mats)


def _upconv_body(relu, x_ref, m_ref, w_ref, b_ref, o_ref):
    i = pl.program_id(0)
    m = m_ref[...]
    acc = jnp.zeros((_TILE, w_ref.shape[2]), jnp.float32)
    win = x_ref[pl.ds(i * _TILE + _TILE - 64, _TILE + 128), :]
    for idx, (dh, dw) in enumerate(_DELTAS):
        off = 64 + dh * 56 + dw
        xs = jax.lax.slice_in_dim(win, off, off + _TILE, axis=0)
        mk = None
        if dh == -1:
            mk = m[:, 0:1]
        elif dh == 1:
            mk = m[:, 1:2]
        if dw == -1:
            mw = m[:, 2:3]
            mk = mw if mk is None else mk * mw
        elif dw == 1:
            mw = m[:, 3:4]
            mk = mw if mk is None else mk * mw
        if mk is not None:
            xs = jnp.where(mk > 0.5, xs, 0.0)
        acc = acc + jnp.dot(xs, w_ref[idx], preferred_element_type=jnp.float32)
    acc = acc + b_ref[...]
    if relu:
        acc = jnp.maximum(acc, 0.0)
    o_ref[...] = acc


def _upconv_pallas(x_flat, masks, w_all, bias, out_ch, relu, pad_out,
                   interpret=False):
    n_total = x_flat.shape[0] - 2 * _TILE
    n_steps = n_total // _TILE
    in_ch = x_flat.shape[1]
    out_rows = n_total + 2 * _TILE if pad_out else n_total
    row_off = 1 if pad_out else 0
    return pl.pallas_call(
        functools.partial(_upconv_body, relu),
        grid=(n_steps,),
        in_specs=[
            pl.BlockSpec((x_flat.shape[0], in_ch), lambda i: (0, 0)),
            pl.BlockSpec((_TILE, 8), lambda i: (i, 0)),
            pl.BlockSpec((9, in_ch, out_ch), lambda i: (0, 0, 0)),
            pl.BlockSpec((1, out_ch), lambda i: (0, 0)),
        ],
        out_specs=pl.BlockSpec((_TILE, out_ch),
                               lambda i, _o=row_off: (i + _o, 0)),
        out_shape=jax.ShapeDtypeStruct((out_rows, out_ch), jnp.float32),
        scratch_shapes=[],
        interpret=interpret,
    )(x_flat, masks, w_all, bias)


def kernel(x, params):
    p = params['proj']
    h = jax.nn.relu(_conv2d(x, p['c1_w'], p['c1_b'], 2, 1))
    h = jax.nn.relu(_conv2d(h, p['c2_w'], p['c2_b'], 2, 1))
    h = _conv2d(h, p['c3_w'], p['c3_b'], 1, 1)
    h = _residual_stack(h, p['res'])
    z = _conv2d(h, params['pre_vq_w'], params['pre_vq_b'], 1, 0)

    n, c, hh, ww = z.shape
    z_flat = jnp.transpose(z, (0, 2, 3, 1)).reshape(-1, c)
    i0, loss, perp = _vq_pallas(z_flat, params['codebook'])
    q_flat = _sc_gather(params['codebook'], i0.reshape(-1))

    d = params['dec']
    qn = q_flat.reshape(n, hh, ww, c)
    r = _conv2d_nhwc(qn, d['c1_w'], d['c1_b'], 1, 1)
    r = _residual_stack_nhwc(r, d['res'])

    masks = jnp.asarray(_upconv_masks())
    r_pad = jnp.pad(r.reshape(-1, 128), ((_TILE, _TILE), (0, 0)))
    y_pad = _upconv_pallas(r_pad, masks, _t1_weights(d['t1_w']),
                           jnp.tile(d['t1_b'], 4)[None, :], 256, True, True)
    out48 = _upconv_pallas(y_pad, masks, _t2_weights(d['t2_w']),
                           jnp.tile(d['t2_b'], 16)[None, :], 48, False, False)
    x_recon = out48.reshape(16, 56, 56, 4, 4, 3)
    x_recon = jnp.transpose(x_recon, (0, 5, 1, 3, 2, 4)).reshape(16, 3, 224, 224)
    return loss, x_recon, perp


# speedup vs baseline: 2.6676x; 2.6676x over previous
import functools

import jax
import jax.numpy as jnp
import numpy as np
from jax.experimental import pallas as pl
from jax.experimental.pallas import tpu as pltpu

_EPS_BN = 1e-5
_K = 1024
_D = 128
_TILE = 1024


def _bn(x):
    return x / np.sqrt(1.0 + _EPS_BN)


def _conv2d(x, w, b=None, stride=1, padding=0):
    out = jax.lax.conv_general_dilated(
        x, w, (stride, stride), [(padding, padding), (padding, padding)],
        dimension_numbers=('NCHW', 'OIHW', 'NCHW'))
    if b is not None:
        out = out + b[None, :, None, None]
    return out


def _conv_transpose2d(x, w, b, stride=2, padding=1):
    kh, kw = w.shape[2], w.shape[3]
    w_t = jnp.transpose(w[:, :, ::-1, ::-1], (1, 0, 2, 3))
    ph, pw = kh - 1 - padding, kw - 1 - padding
    out = jax.lax.conv_general_dilated(
        x, w_t, (1, 1), [(ph, ph), (pw, pw)], lhs_dilation=(stride, stride),
        dimension_numbers=('NCHW', 'OIHW', 'NCHW'))
    return out + b[None, :, None, None]


def _residual_stack(x, layers):
    for p in layers:
        h = jax.nn.relu(x)
        h = _conv2d(h, p['w1'], None, 1, 1)
        h = _bn(h)
        h = jax.nn.relu(h)
        h = _conv2d(h, p['w2'], None, 1, 0)
        h = _bn(h)
        x = x + h
    return jax.nn.relu(x)


def _conv2d_nhwc(x, w, b=None, stride=1, padding=0):
    out = jax.lax.conv_general_dilated(
        x, jnp.transpose(w, (2, 3, 1, 0)), (stride, stride),
        [(padding, padding), (padding, padding)],
        dimension_numbers=('NHWC', 'HWIO', 'NHWC'))
    if b is not None:
        out = out + b[None, None, None, :]
    return out


def _residual_stack_nhwc(x, layers):
    for p in layers:
        h = jax.nn.relu(x)
        h = _conv2d_nhwc(h, p['w1'], None, 1, 1)
        h = _bn(h)
        h = jax.nn.relu(h)
        h = _conv2d_nhwc(h, p['w2'], None, 1, 0)
        h = _bn(h)
        x = x + h
    return jax.nn.relu(x)


def _vq_body(n_total, n_steps, z_ref, mcbt_ref, c2_ref, cbt_ref, q_ref,
             loss_ref, perp_ref, hist_ref, acc_ref):
    i = pl.program_id(0)

    @pl.when(i == 0)
    def _():
        hist_ref[...] = jnp.zeros_like(hist_ref)
        acc_ref[0, 0] = 0.0

    zt = z_ref[...]
    dist = jnp.dot(zt, mcbt_ref[...],
                   preferred_element_type=jnp.float32) + c2_ref[...]

    iota = jax.lax.broadcasted_iota(jnp.int32, dist.shape, 1)
    big = jnp.float32(jnp.inf)

    d0 = jnp.min(dist, axis=1, keepdims=True)
    i0 = jnp.min(jnp.where(dist == d0, iota, _K), axis=1, keepdims=True)
    dist1 = jnp.where(iota == i0, big, dist)
    d1 = jnp.min(dist1, axis=1, keepdims=True)
    i1 = jnp.min(jnp.where(dist1 == d1, iota, _K), axis=1, keepdims=True)
    dist2 = jnp.where(iota == i1, big, dist1)
    d2 = jnp.min(dist2, axis=1, keepdims=True)
    i2 = jnp.min(jnp.where(dist2 == d2, iota, _K), axis=1, keepdims=True)

    onehot0 = (iota == i0).astype(jnp.float32)
    q = jax.lax.dot_general(onehot0, cbt_ref[...], (((1,), (1,)), ((), ())),
                            preferred_element_type=jnp.float32)
    q_ref[...] = q

    z2 = jnp.sum(zt * zt, axis=1, keepdims=True)
    acc_ref[0, 0] += jnp.sum(d0 + z2)

    onehot2 = (iota == i2).astype(jnp.float32)
    hist_ref[...] += jnp.sum(onehot2, axis=0, keepdims=True)

    @pl.when(i == n_steps - 1)
    def _():
        loss_ref[0, 0] = acc_ref[0, 0] * (0.25 / (n_total * _D))
        avg = hist_ref[...] / n_total
        perp_ref[0, 0] = jnp.exp(-jnp.sum(avg * jnp.log(avg + 1e-10)))


@functools.partial(jax.jit, static_argnames=('interpret',))
def _vq_pallas(z_flat, codebook, interpret=False):
    n_total = z_flat.shape[0]
    n_steps = n_total // _TILE
    cbt = codebook.T
    mcbt = -2.0 * cbt
    c2 = jnp.sum(codebook * codebook, axis=1)[None, :]
    q, loss, perp = pl.pallas_call(
        functools.partial(_vq_body, n_total, n_steps),
        grid=(n_steps,),
        in_specs=[
            pl.BlockSpec((_TILE, _D), lambda i: (i, 0)),
            pl.BlockSpec((_D, _K), lambda i: (0, 0)),
            pl.BlockSpec((1, _K), lambda i: (0, 0)),
            pl.BlockSpec((_D, _K), lambda i: (0, 0)),
        ],
        out_specs=[
            pl.BlockSpec((_TILE, _D), lambda i: (i, 0)),
            pl.BlockSpec(memory_space=pltpu.SMEM),
            pl.BlockSpec(memory_space=pltpu.SMEM),
        ],
        out_shape=[
            jax.ShapeDtypeStruct((n_total, _D), jnp.float32),
            jax.ShapeDtypeStruct((1, 1), jnp.float32),
            jax.ShapeDtypeStruct((1, 1), jnp.float32),
        ],
        scratch_shapes=[
            pltpu.VMEM((1, _K), jnp.float32),
            pltpu.SMEM((1, 1), jnp.float32),
        ],
        interpret=interpret,
    )(z_flat, mcbt, c2, cbt)
    return q, loss[0, 0], perp[0, 0]


_DELTAS = tuple((dh, dw) for dh in (-1, 0, 1) for dw in (-1, 0, 1))
_KMAP = {(0, -1): 0, (0, 0): 2, (1, 0): 1, (1, 1): 3}
_T2TAB = (
    ((1, -1, 0), (0, 0, 2)),
    ((0, 0, 1), (1, 0, 3)),
    ((0, 0, 0), (1, 0, 2)),
    ((1, 0, 1), (0, 1, 3)),
)


def _upconv_masks():
    g = np.arange(50176)
    h = (g // 56) % 56
    w = g % 56
    cols = [h > 0, h < 55, w > 0, w < 55, np.ones(50176, bool),
            np.ones(50176, bool), np.ones(50176, bool), np.ones(50176, bool)]
    return np.stack(cols, axis=1).astype(np.float32)


def _t1_weights(t1_w):
    w_t = jnp.transpose(t1_w[:, :, ::-1, ::-1], (1, 0, 2, 3))
    mats = []
    for dh, dw in _DELTAS:
        wd = jnp.zeros((128, 256), jnp.float32)
        for rh in (0, 1):
            for rw in (0, 1):
                if (rh, dh) in _KMAP and (rw, dw) in _KMAP:
                    blk = (rh * 2 + rw) * 64
                    wd = wd.at[:, blk:blk + 64].set(
                        w_t[:, :, _KMAP[(rh, dh)], _KMAP[(rw, dw)]].T)
        mats.append(wd)
    return jnp.stack(mats)


def _t2_weights(t2_w):
    v_t = jnp.transpose(t2_w[:, :, ::-1, ::-1], (1, 0, 2, 3))
    mats = []
    for dh, dw in _DELTAS:
        vd = jnp.zeros((256, 48), jnp.float32)
        for ch in range(4):
            for cw in range(4):
                for (rho_h, dh2, vrh) in _T2TAB[ch]:
                    if dh2 != dh:
                        continue
                    for (rho_w, dw2, vrw) in _T2TAB[cw]:
                        if dw2 != dw:
                            continue
                        rblk = (rho_h * 2 + rho_w) * 64
                        cblk = (ch * 4 + cw) * 3
                        vd = vd.at[rblk:rblk + 64, cblk:cblk + 3].add(
                            v_t[:, :, vrh, vrw].T)
        mats.append(vd)
    return jnp.stack(mats)


def _upconv_body(relu, x_ref, m_ref, w_ref, b_ref, o_ref):
    i = pl.program_id(0)
    m = m_ref[...]
    acc = jnp.zeros((_TILE, w_ref.shape[2]), jnp.float32)
    win = x_ref[pl.ds(i * _TILE + _TILE - 64, _TILE + 128), :]
    for idx, (dh, dw) in enumerate(_DELTAS):
        off = 64 + dh * 56 + dw
        xs = jax.lax.slice_in_dim(win, off, off + _TILE, axis=0)
        mk = None
        if dh == -1:
            mk = m[:, 0:1]
        elif dh == 1:
            mk = m[:, 1:2]
        if dw == -1:
            mw = m[:, 2:3]
            mk = mw if mk is None else mk * mw
        elif dw == 1:
            mw = m[:, 3:4]
            mk = mw if mk is None else mk * mw
        if mk is not None:
            xs = jnp.where(mk > 0.5, xs, 0.0)
        acc = acc + jnp.dot(xs, w_ref[idx], preferred_element_type=jnp.float32)
    acc = acc + b_ref[...]
    if relu:
        acc = jnp.maximum(acc, 0.0)
    o_ref[...] = acc


def _upconv_pallas(x_flat, masks, w_all, bias, out_ch, relu, pad_out,
                   interpret=False):
    n_total = x_flat.shape[0] - 2 * _TILE
    n_steps = n_total // _TILE
    in_ch = x_flat.shape[1]
    out_rows = n_total + 2 * _TILE if pad_out else n_total
    row_off = 1 if pad_out else 0
    return pl.pallas_call(
        functools.partial(_upconv_body, relu),
        grid=(n_steps,),
        in_specs=[
            pl.BlockSpec((x_flat.shape[0], in_ch), lambda i: (0, 0)),
            pl.BlockSpec((_TILE, 8), lambda i: (i, 0)),
            pl.BlockSpec((9, in_ch, out_ch), lambda i: (0, 0, 0)),
            pl.BlockSpec((1, out_ch), lambda i: (0, 0)),
        ],
        out_specs=pl.BlockSpec((_TILE, out_ch),
                               lambda i, _o=row_off: (i + _o, 0)),
        out_shape=jax.ShapeDtypeStruct((out_rows, out_ch), jnp.float32),
        scratch_shapes=[],
        interpret=interpret,
    )(x_flat, masks, w_all, bias)


def kernel(x, params):
    p = params['proj']
    h = jax.nn.relu(_conv2d(x, p['c1_w'], p['c1_b'], 2, 1))
    h = jax.nn.relu(_conv2d(h, p['c2_w'], p['c2_b'], 2, 1))
    h = _conv2d(h, p['c3_w'], p['c3_b'], 1, 1)
    h = _residual_stack(h, p['res'])
    z = _conv2d(h, params['pre_vq_w'], params['pre_vq_b'], 1, 0)

    n, c, hh, ww = z.shape
    z_flat = jnp.transpose(z, (0, 2, 3, 1)).reshape(-1, c)
    q_flat, loss, perp = _vq_pallas(z_flat, params['codebook'])

    d = params['dec']
    qn = q_flat.reshape(n, hh, ww, c)
    r = _conv2d_nhwc(qn, d['c1_w'], d['c1_b'], 1, 1)
    r = _residual_stack_nhwc(r, d['res'])

    masks = jnp.asarray(_upconv_masks())
    r_pad = jnp.pad(r.reshape(-1, 128), ((_TILE, _TILE), (0, 0)))
    y_pad = _upconv_pallas(r_pad, masks, _t1_weights(d['t1_w']),
                           jnp.tile(d['t1_b'], 4)[None, :], 256, True, True)
    out48 = _upconv_pallas(y_pad, masks, _t2_weights(d['t2_w']),
                           jnp.tile(d['t2_b'], 16)[None, :], 48, False, False)
    x_recon = out48.reshape(16, 56, 56, 4, 4, 3)
    x_recon = jnp.transpose(x_recon, (0, 5, 1, 3, 2, 4)).reshape(16, 3, 224, 224)
    return loss, x_recon, perp
